# baseline (speedup 1.0000x reference)
import jax
import jax.numpy as jnp
from jax import lax
from jax.experimental import pallas as pl
from jax.experimental.pallas import tpu as pltpu
from jax.experimental.pallas import tpu_sc as plsc

B = 4096
C_IN = 19
C_OUT = 64
D = 128
NC = 2
NS = 16
NW = NC * NS
ROWS_PER_W = B // NW
CHUNK = 8
N_CHUNKS = ROWS_PER_W // CHUNK


NBUF = 3
LOOKAHEAD = 3
C_LO = 24
N_HI = C_OUT - C_LO


def _sc_body(inp_hbm, maskblk_hbm, maskpad_hbm, out_hbm, mask_v, ring_v,
             sem_in, sem_wout, sem_mask, sem_fill):
    wid = lax.axis_index("s") * NC + lax.axis_index("c")
    base = wid * ROWS_PER_W
    mask_fill = pltpu.async_copy(maskblk_hbm, mask_v, sem_fill)
    pad_fill = [
        pltpu.async_copy(maskpad_hbm,
                         ring_v.at[b].at[:, pl.ds(C_IN, C_LO - C_IN)], sem_fill)
        for b in range(NBUF)
    ]

    in_dma = [None] * N_CHUNKS
    wout_dma = [None] * N_CHUNKS
    wmask_dma = [None] * N_CHUNKS

    def start_in(c):
        in_dma[c] = pltpu.async_copy(
            inp_hbm.at[pl.ds(base + c * CHUNK, CHUNK)],
            ring_v.at[c % NBUF].at[:, pl.ds(0, C_IN)], sem_in)

    for c in range(LOOKAHEAD):
        start_in(c)
    mask_fill.wait()
    for b in range(NBUF):
        pad_fill[b].wait()
    for c in range(N_CHUNKS):
        r0 = base + c * CHUNK
        wmask_dma[c] = pltpu.async_copy(
            mask_v, out_hbm.at[pl.ds(r0, CHUNK), pl.ds(C_LO, N_HI)],
            sem_mask)
        in_dma[c].wait()
        wout_dma[c] = pltpu.async_copy(
            ring_v.at[c % NBUF], out_hbm.at[pl.ds(r0, CHUNK), pl.ds(0, C_LO)],
            sem_wout)
        if c >= LOOKAHEAD:
            wmask_dma[c - LOOKAHEAD].wait()
        nxt = c + LOOKAHEAD
        if nxt < N_CHUNKS:
            if nxt >= NBUF:
                wout_dma[nxt - NBUF].wait()
            start_in(nxt)
    for c in range(max(0, N_CHUNKS - NBUF), N_CHUNKS):
        wout_dma[c].wait()
    for c in range(max(0, N_CHUNKS - LOOKAHEAD), N_CHUNKS):
        wmask_dma[c].wait()


_sc_call = pl.kernel(
    _sc_body,
    mesh=plsc.VectorSubcoreMesh(core_axis_name="c", subcore_axis_name="s"),
    out_type=jax.ShapeDtypeStruct((B, C_OUT, D), jnp.float32),
    scratch_types=[
        pltpu.VMEM((CHUNK, N_HI, D), jnp.float32),
        pltpu.VMEM((NBUF, CHUNK, C_LO, D), jnp.float32),
        pltpu.SemaphoreType.DMA,
        pltpu.SemaphoreType.DMA,
        pltpu.SemaphoreType.DMA,
        pltpu.SemaphoreType.DMA,
    ],
)


@jax.jit
def kernel(inp, mask_token):
    mrow = mask_token.reshape(1, 1, D)
    maskblk = jnp.broadcast_to(mrow, (CHUNK, N_HI, D))
    maskpad = jnp.broadcast_to(mrow, (CHUNK, C_LO - C_IN, D))
    return _sc_call(inp, maskblk, maskpad)


# speedup vs baseline: 1.0059x; 1.0059x over previous
import jax
import jax.numpy as jnp
from jax import lax
from jax.experimental import pallas as pl
from jax.experimental.pallas import tpu as pltpu
from jax.experimental.pallas import tpu_sc as plsc

B = 4096
C_IN = 19
C_OUT = 64
D = 128
NC = 2
NS = 16
NW = NC * NS
ROWS_PER_W = B // NW
CHUNK = 8
N_CHUNKS = ROWS_PER_W // CHUNK


NBUF = 3
LOOKAHEAD = 3
C_LO = 24
N_HI = C_OUT - C_LO


def _sc_body(inp_hbm, maskblk_hbm, maskpad_hbm, out_hbm, mask_v, ring_v,
             sem_in, sem_wout, sem_mask, sem_fill):
    wid = lax.axis_index("s") * NC + lax.axis_index("c")
    base = wid * ROWS_PER_W
    mask_fill = pltpu.async_copy(maskblk_hbm, mask_v, sem_fill)
    pad_fill = [
        pltpu.async_copy(maskpad_hbm,
                         ring_v.at[b].at[:, pl.ds(C_IN, C_LO - C_IN)], sem_fill)
        for b in range(NBUF)
    ]

    in_dma = [None] * N_CHUNKS
    wout_dma = [None] * N_CHUNKS
    wmask_dma = [None] * N_CHUNKS

    def start_in(c):
        in_dma[c] = pltpu.async_copy(
            inp_hbm.at[pl.ds(base + c * CHUNK, CHUNK)],
            ring_v.at[c % NBUF].at[:, pl.ds(0, C_IN)], sem_in)

    for c in range(LOOKAHEAD):
        start_in(c)
    mask_fill.wait()
    for b in range(NBUF):
        pad_fill[b].wait()
    for c in range(N_CHUNKS):
        r0 = base + c * CHUNK
        wmask_dma[c] = pltpu.async_copy(
            mask_v, out_hbm.at[pl.ds(r0, CHUNK), pl.ds(C_LO, N_HI)],
            sem_mask)
        in_dma[c].wait()
        wout_dma[c] = pltpu.async_copy(
            ring_v.at[c % NBUF], out_hbm.at[pl.ds(r0, CHUNK), pl.ds(0, C_LO)],
            sem_wout)
        if c >= 2 * LOOKAHEAD:
            wmask_dma[c - 2 * LOOKAHEAD].wait()
        nxt = c + LOOKAHEAD
        if nxt < N_CHUNKS:
            if nxt >= NBUF:
                wout_dma[nxt - NBUF].wait()
            start_in(nxt)
    for c in range(max(0, N_CHUNKS - NBUF), N_CHUNKS):
        wout_dma[c].wait()
    for c in range(max(0, N_CHUNKS - 2 * LOOKAHEAD), N_CHUNKS):
        wmask_dma[c].wait()


_sc_call = pl.kernel(
    _sc_body,
    mesh=plsc.VectorSubcoreMesh(core_axis_name="c", subcore_axis_name="s"),
    out_type=jax.ShapeDtypeStruct((B, C_OUT, D), jnp.float32),
    scratch_types=[
        pltpu.VMEM((CHUNK, N_HI, D), jnp.float32),
        pltpu.VMEM((NBUF, CHUNK, C_LO, D), jnp.float32),
        pltpu.SemaphoreType.DMA,
        pltpu.SemaphoreType.DMA,
        pltpu.SemaphoreType.DMA,
        pltpu.SemaphoreType.DMA,
    ],
)


@jax.jit
def kernel(inp, mask_token):
    mrow = mask_token.reshape(1, 1, D)
    maskblk = jnp.broadcast_to(mrow, (CHUNK, N_HI, D))
    maskpad = jnp.broadcast_to(mrow, (CHUNK, C_LO - C_IN, D))
    return _sc_call(inp, maskblk, maskpad)
